# baseline (speedup 1.0000x reference)
import functools

import jax
import jax.numpy as jnp
from jax import lax
from jax.experimental import pallas as pl
from jax.experimental.pallas import tpu as pltpu
from jax.experimental.pallas import tpu_sc as plsc

N = 2048
H = 8
DH = 64
F = 512
NIT = 8
TOPK = 10
BLK = 256
NNCLS = 64


def _proj_body(fea_ref, wq_ref, bq_ref, wk_ref, bk_ref, qs_ref, k_ref):
    fea = fea_ref[...]
    dn = (((1,), (1,)), ((), ()))
    q = jax.lax.dot_general(fea, wq_ref[...], dn,
                            preferred_element_type=jnp.float32) + bq_ref[...]
    k = jax.lax.dot_general(fea, wk_ref[...], dn,
                            preferred_element_type=jnp.float32) + bk_ref[...]
    s = jnp.sqrt(jnp.sum(q * q) * jnp.sum(k * k))
    ks_sum = jnp.sum(k, axis=0, keepdims=True)
    scales = []
    for h in range(H):
        sl = slice(h * DH, (h + 1) * DH)
        dh = jnp.sum(q[:, sl] * ks_sum[:, sl], axis=1, keepdims=True)
        c = 1.0 / (H * (dh + N * s))
        scales.append(jnp.broadcast_to(c, (N, DH)))
    qs_ref[...] = q * jnp.concatenate(scales, axis=1)
    k_ref[...] = k


def _attn_body(qs_ref, kf_ref, gum_ref, gs_ref):
    logits = jax.lax.dot_general(qs_ref[...], kf_ref[...],
                                 (((1,), (1,)), ((), ())),
                                 preferred_element_type=jnp.float32)
    logits = logits + gum_ref[...]
    m = jnp.max(logits, axis=1, keepdims=True)
    e = jnp.exp(logits - m)
    gs_ref[...] = e / jnp.sum(e, axis=1, keepdims=True)


SC_BATCH = 16
SC_NW = 32
SC_RPW = N // SC_NW
F32_MIN = float(jnp.finfo(jnp.float32).min)


UNROLL = 8


def _sc_topk_body(gs_hbm, attn_hbm, buf, sem):
    c = lax.axis_index("c")
    s = lax.axis_index("s")
    wid = s * 2 + c
    lowest = jnp.full((16,), F32_MIN, jnp.float32)
    lane = lax.iota(jnp.int32, 16)

    ROWG = 4

    def do_rows(r, carry):
        bases = [r * ROWG + v for v in range(ROWG)]

        def chunk_step(g, ts):
            for u in range(UNROLL):
                tsn = []
                for v in range(ROWG):
                    t = ts[v]
                    x = buf[bases[v], pl.ds((g * UNROLL + u) * 16, 16)]
                    tl = [jnp.maximum(x, t[0])] + list(t[1:])
                    for j in range(TOPK - 1):
                        hi = jnp.maximum(tl[j], tl[j + 1])
                        lo = jnp.minimum(tl[j], tl[j + 1])
                        tl[j], tl[j + 1] = lo, hi
                    tsn.append(tuple(tl))
                ts = tuple(tsn)
            return ts

        ts = lax.fori_loop(0, N // 16 // UNROLL, chunk_step,
                           (((lowest,) * TOPK),) * ROWG)

        thetas = []
        for v in range(ROWG):
            tl = list(ts[v])
            theta = None
            for _ in range(TOPK):
                m = jnp.max(tl[TOPK - 1])
                win1 = jnp.min(jnp.where(tl[TOPK - 1] == m, lane, 16))
                sel = lane == win1
                for j in range(TOPK - 1, 0, -1):
                    tl[j] = jnp.where(sel, tl[j - 1], tl[j])
                tl[0] = jnp.where(sel, lowest, tl[0])
                theta = m
            thetas.append(theta)

        def mask_step(g, carry):
            for u in range(UNROLL):
                for v in range(ROWG):
                    off = (g * UNROLL + u) * 16
                    x = buf[bases[v], pl.ds(off, 16)]
                    buf[bases[v], pl.ds(off, 16)] = jnp.where(
                        x >= thetas[v], x, 0.0)
            return carry

        return lax.fori_loop(0, N // 16 // UNROLL, mask_step, carry)

    for b in range(SC_RPW // SC_BATCH):
        row0 = wid * SC_RPW + b * SC_BATCH
        pltpu.sync_copy(gs_hbm.at[pl.ds(row0, SC_BATCH), :], buf)
        lax.fori_loop(0, SC_BATCH // ROWG, do_rows, 0)
        pltpu.sync_copy(buf, attn_hbm.at[pl.ds(row0, SC_BATCH), :])


_sc_topk = pl.kernel(
    _sc_topk_body,
    out_type=jax.ShapeDtypeStruct((N, N), jnp.float32),
    mesh=plsc.VectorSubcoreMesh(core_axis_name="c", subcore_axis_name="s"),
    scratch_types=[pltpu.VMEM((SC_BATCH, N), jnp.float32),
                   pltpu.SemaphoreType.DMA],
    compiler_params=pltpu.CompilerParams(use_tc_tiling_on_sc=False,
                                         needs_layout_passes=False),
)


def _horner_body(a_ref, ah_ref, preds_ref, b2_ref, b1_ref, out_ref):
    dn = (((1,), (0,)), ((), ()))
    a = a_ref[...]
    tmp = preds_ref[...]
    acc = tmp * b2_ref[0, 0]
    for i in range(1, NIT):
        tmp = jax.lax.dot_general(a, tmp, dn, preferred_element_type=jnp.float32)
        tmp = jax.lax.dot_general(a, tmp, dn, preferred_element_type=jnp.float32)
        acc = acc + tmp * b2_ref[0, i]
    ah = ah_ref[...]
    tmp = acc
    acc = tmp * b1_ref[0, 0]
    for i in range(1, NIT):
        tmp = jax.lax.dot_general(ah, tmp, dn, preferred_element_type=jnp.float32)
        acc = acc + tmp * b1_ref[0, i]
    out_ref[...] = acc


def kernel(local_preds, idx, origin_fea, A_hat, Wq_w, Wq_b, Wk_w, Wk_b,
           lin1_w, lin2_w, gumbel):
    f32 = jnp.float32
    bq = Wq_b.reshape(1, F)
    bk = Wk_b.reshape(1, F)

    qs, k = pl.pallas_call(
        _proj_body,
        out_shape=[jax.ShapeDtypeStruct((N, F), f32),
                   jax.ShapeDtypeStruct((N, F), f32)],
    )(origin_fea, Wq_w, bq, Wk_w, bk)

    nblk = N // BLK
    gs = pl.pallas_call(
        _attn_body,
        grid=(nblk,),
        in_specs=[pl.BlockSpec((BLK, F), lambda i: (i, 0)),
                  pl.BlockSpec((N, F), lambda i: (0, 0)),
                  pl.BlockSpec((BLK, N), lambda i: (i, 0))],
        out_specs=pl.BlockSpec((BLK, N), lambda i: (i, 0)),
        out_shape=jax.ShapeDtypeStruct((N, N), f32),
    )(qs, k, gumbel)

    attn = _sc_topk(gs)

    out = pl.pallas_call(
        _horner_body,
        in_specs=[pl.BlockSpec(memory_space=pltpu.MemorySpace.VMEM),
                  pl.BlockSpec(memory_space=pltpu.MemorySpace.VMEM),
                  pl.BlockSpec(memory_space=pltpu.MemorySpace.VMEM),
                  pl.BlockSpec(memory_space=pltpu.MemorySpace.SMEM),
                  pl.BlockSpec(memory_space=pltpu.MemorySpace.SMEM)],
        out_shape=jax.ShapeDtypeStruct((N, NNCLS), f32),
    )(attn, A_hat, local_preds, lin2_w, lin1_w)
    return out


# speedup vs baseline: 1.3100x; 1.3100x over previous
import functools

import jax
import jax.numpy as jnp
from jax import lax
from jax.experimental import pallas as pl
from jax.experimental.pallas import tpu as pltpu
from jax.experimental.pallas import tpu_sc as plsc

N = 2048
H = 8
DH = 64
F = 512
NIT = 8
TOPK = 10
BLK = 256
NNCLS = 64


def _proj_body(fea_ref, wq_ref, bq_ref, wk_ref, bk_ref, qs_ref, k_ref):
    fea = fea_ref[...]
    dn = (((1,), (1,)), ((), ()))
    q = jax.lax.dot_general(fea, wq_ref[...], dn,
                            preferred_element_type=jnp.float32) + bq_ref[...]
    k = jax.lax.dot_general(fea, wk_ref[...], dn,
                            preferred_element_type=jnp.float32) + bk_ref[...]
    s = jnp.sqrt(jnp.sum(q * q) * jnp.sum(k * k))
    ks_sum = jnp.sum(k, axis=0, keepdims=True)
    scales = []
    for h in range(H):
        sl = slice(h * DH, (h + 1) * DH)
        dh = jnp.sum(q[:, sl] * ks_sum[:, sl], axis=1, keepdims=True)
        c = 1.0 / (H * (dh + N * s))
        scales.append(jnp.broadcast_to(c, (N, DH)))
    qs_ref[...] = q * jnp.concatenate(scales, axis=1)
    k_ref[...] = k


def _attn_body(qs_ref, kf_ref, gum_ref, gs_ref):
    logits = jax.lax.dot_general(qs_ref[...], kf_ref[...],
                                 (((1,), (1,)), ((), ())),
                                 preferred_element_type=jnp.float32)
    logits = logits + gum_ref[...]
    m = jnp.max(logits, axis=1, keepdims=True)
    e = jnp.exp(logits - m)
    gs_ref[...] = e / jnp.sum(e, axis=1, keepdims=True)


def _topk_mask_body(gs_ref, attn_ref):
    gs = gs_ref[...]
    colid = jax.lax.broadcasted_iota(jnp.int32, gs.shape, 1)
    work = gs
    sel = jnp.zeros(gs.shape, dtype=jnp.bool_)
    for _ in range(TOPK):
        mx = jnp.max(work, axis=1, keepdims=True)
        cand = jnp.where(work == mx, colid, N)
        chosen = colid == jnp.min(cand, axis=1, keepdims=True)
        sel = jnp.logical_or(sel, chosen)
        work = jnp.where(chosen, -jnp.inf, work)
    attn_ref[...] = jnp.where(sel, gs, 0.0)


SC_NW = 32
SC_ROWS = 256
SPLIT = N - SC_ROWS
SC_RPW = SC_ROWS // SC_NW
SC_BATCH = SC_RPW
F32_MIN = float(jnp.finfo(jnp.float32).min)

UNROLL = 8


def _sc_topk_body(gs_hbm, attn_hbm, buf, sem):
    c = lax.axis_index("c")
    s = lax.axis_index("s")
    wid = s * 2 + c
    lowest = jnp.full((16,), F32_MIN, jnp.float32)
    lane = lax.iota(jnp.int32, 16)

    ROWG = 4

    def do_rows(r, carry):
        bases = [r * ROWG + v for v in range(ROWG)]

        def chunk_step(g, ts):
            for u in range(UNROLL):
                tsn = []
                for v in range(ROWG):
                    t = ts[v]
                    x = buf[bases[v], pl.ds((g * UNROLL + u) * 16, 16)]
                    tl = [jnp.maximum(x, t[0])] + list(t[1:])
                    for j in range(TOPK - 1):
                        hi = jnp.maximum(tl[j], tl[j + 1])
                        lo = jnp.minimum(tl[j], tl[j + 1])
                        tl[j], tl[j + 1] = lo, hi
                    tsn.append(tuple(tl))
                ts = tuple(tsn)
            return ts

        ts = lax.fori_loop(0, N // 16 // UNROLL, chunk_step,
                           (((lowest,) * TOPK),) * ROWG)

        thetas = []
        for v in range(ROWG):
            tl = list(ts[v])
            theta = None
            for _ in range(TOPK):
                m = jnp.max(tl[TOPK - 1])
                win1 = jnp.min(jnp.where(tl[TOPK - 1] == m, lane, 16))
                sel = lane == win1
                for j in range(TOPK - 1, 0, -1):
                    tl[j] = jnp.where(sel, tl[j - 1], tl[j])
                tl[0] = jnp.where(sel, lowest, tl[0])
                theta = m
            thetas.append(theta)

        def mask_step(g, carry):
            for u in range(UNROLL):
                for v in range(ROWG):
                    off = (g * UNROLL + u) * 16
                    x = buf[bases[v], pl.ds(off, 16)]
                    buf[bases[v], pl.ds(off, 16)] = jnp.where(
                        x >= thetas[v], x, 0.0)
            return carry

        return lax.fori_loop(0, N // 16 // UNROLL, mask_step, carry)

    row0 = wid * SC_RPW
    pltpu.sync_copy(gs_hbm.at[pl.ds(SPLIT + row0, SC_BATCH), :], buf)
    lax.fori_loop(0, SC_BATCH // ROWG, do_rows, 0)
    pltpu.sync_copy(buf, attn_hbm.at[pl.ds(row0, SC_BATCH), :])


_sc_topk = pl.kernel(
    _sc_topk_body,
    out_type=jax.ShapeDtypeStruct((SC_ROWS, N), jnp.float32),
    mesh=plsc.VectorSubcoreMesh(core_axis_name="c", subcore_axis_name="s"),
    scratch_types=[pltpu.VMEM((SC_BATCH, N), jnp.float32),
                   pltpu.SemaphoreType.DMA],
    compiler_params=pltpu.CompilerParams(use_tc_tiling_on_sc=False,
                                         needs_layout_passes=False),
)


def _horner_body(a1_ref, a2_ref, ah_ref, preds_ref, b2_ref, b1_ref, out_ref):
    dn = (((1,), (0,)), ((), ()))
    a = jnp.concatenate([a1_ref[...], a2_ref[...]], axis=0)
    tmp = preds_ref[...]
    acc = tmp * b2_ref[0, 0]
    for i in range(1, NIT):
        tmp = jax.lax.dot_general(a, tmp, dn, preferred_element_type=jnp.float32)
        tmp = jax.lax.dot_general(a, tmp, dn, preferred_element_type=jnp.float32)
        acc = acc + tmp * b2_ref[0, i]
    ah = ah_ref[...]
    tmp = acc
    acc = tmp * b1_ref[0, 0]
    for i in range(1, NIT):
        tmp = jax.lax.dot_general(ah, tmp, dn, preferred_element_type=jnp.float32)
        acc = acc + tmp * b1_ref[0, i]
    out_ref[...] = acc


def kernel(local_preds, idx, origin_fea, A_hat, Wq_w, Wq_b, Wk_w, Wk_b,
           lin1_w, lin2_w, gumbel):
    f32 = jnp.float32
    bq = Wq_b.reshape(1, F)
    bk = Wk_b.reshape(1, F)

    qs, k = pl.pallas_call(
        _proj_body,
        out_shape=[jax.ShapeDtypeStruct((N, F), f32),
                   jax.ShapeDtypeStruct((N, F), f32)],
    )(origin_fea, Wq_w, bq, Wk_w, bk)

    nblk = N // BLK
    gs = pl.pallas_call(
        _attn_body,
        grid=(nblk,),
        in_specs=[pl.BlockSpec((BLK, F), lambda i: (i, 0)),
                  pl.BlockSpec((N, F), lambda i: (0, 0)),
                  pl.BlockSpec((BLK, N), lambda i: (i, 0))],
        out_specs=pl.BlockSpec((BLK, N), lambda i: (i, 0)),
        out_shape=jax.ShapeDtypeStruct((N, N), f32),
    )(qs, k, gumbel)

    attn_bot = _sc_topk(gs)

    attn_top = pl.pallas_call(
        _topk_mask_body,
        grid=(SPLIT // BLK,),
        in_specs=[pl.BlockSpec((BLK, N), lambda i: (i, 0))],
        out_specs=pl.BlockSpec((BLK, N), lambda i: (i, 0)),
        out_shape=jax.ShapeDtypeStruct((SPLIT, N), f32),
    )(gs)

    out = pl.pallas_call(
        _horner_body,
        in_specs=[pl.BlockSpec(memory_space=pltpu.MemorySpace.VMEM),
                  pl.BlockSpec(memory_space=pltpu.MemorySpace.VMEM),
                  pl.BlockSpec(memory_space=pltpu.MemorySpace.VMEM),
                  pl.BlockSpec(memory_space=pltpu.MemorySpace.VMEM),
                  pl.BlockSpec(memory_space=pltpu.MemorySpace.SMEM),
                  pl.BlockSpec(memory_space=pltpu.MemorySpace.SMEM)],
        out_shape=jax.ShapeDtypeStruct((N, NNCLS), f32),
    )(attn_top, attn_bot, A_hat, local_preds, lin2_w, lin1_w)
    return out


# speedup vs baseline: 1.3840x; 1.0565x over previous
import functools

import jax
import jax.numpy as jnp
from jax import lax
from jax.experimental import pallas as pl
from jax.experimental.pallas import tpu as pltpu
from jax.experimental.pallas import tpu_sc as plsc

N = 2048
H = 8
DH = 64
F = 512
NIT = 8
TOPK = 10
BLK = 256
NNCLS = 64


def _proj_body(fea_ref, wq_ref, bq_ref, wk_ref, bk_ref, qs_ref, k_ref):
    fea = fea_ref[...]
    dn = (((1,), (1,)), ((), ()))
    q = jax.lax.dot_general(fea, wq_ref[...], dn,
                            preferred_element_type=jnp.float32) + bq_ref[...]
    k = jax.lax.dot_general(fea, wk_ref[...], dn,
                            preferred_element_type=jnp.float32) + bk_ref[...]
    s = jnp.sqrt(jnp.sum(q * q) * jnp.sum(k * k))
    ks_sum = jnp.sum(k, axis=0, keepdims=True)
    scales = []
    for h in range(H):
        sl = slice(h * DH, (h + 1) * DH)
        dh = jnp.sum(q[:, sl] * ks_sum[:, sl], axis=1, keepdims=True)
        c = 1.0 / (H * (dh + N * s))
        scales.append(jnp.broadcast_to(c, (N, DH)))
    qs_ref[...] = q * jnp.concatenate(scales, axis=1)
    k_ref[...] = k


def _attn_body(qs_ref, kf_ref, gum_ref, gs_ref):
    logits = jax.lax.dot_general(qs_ref[...], kf_ref[...],
                                 (((1,), (1,)), ((), ())),
                                 preferred_element_type=jnp.float32)
    logits = logits + gum_ref[...]
    m = jnp.max(logits, axis=1, keepdims=True)
    e = jnp.exp(logits - m)
    gs_ref[...] = e / jnp.sum(e, axis=1, keepdims=True)


def _topk_mask_body(gs_ref, attn_ref):
    gs = gs_ref[...]
    colid = jax.lax.broadcasted_iota(jnp.int32, gs.shape, 1)
    work = gs
    sel = jnp.zeros(gs.shape, dtype=jnp.bool_)
    for _ in range(TOPK):
        mx = jnp.max(work, axis=1, keepdims=True)
        cand = jnp.where(work == mx, colid, N)
        chosen = colid == jnp.min(cand, axis=1, keepdims=True)
        sel = jnp.logical_or(sel, chosen)
        work = jnp.where(chosen, -jnp.inf, work)
    attn_ref[...] = jnp.where(sel, gs, 0.0)


SC_NW = 32
SC_ROWS = 256
SPLIT = N - SC_ROWS
SC_RPW = SC_ROWS // SC_NW
SC_BATCH = SC_RPW
F32_MIN = float(jnp.finfo(jnp.float32).min)

UNROLL = 8


def _sc_topk_body(gs_hbm, attn_hbm, buf, sem):
    c = lax.axis_index("c")
    s = lax.axis_index("s")
    wid = s * 2 + c
    lowest = jnp.full((16,), F32_MIN, jnp.float32)
    lane = lax.iota(jnp.int32, 16)

    ROWG = 4

    def do_rows(r, carry):
        bases = [r * ROWG + v for v in range(ROWG)]

        def chunk_step(g, ts):
            for u in range(UNROLL):
                tsn = []
                for v in range(ROWG):
                    t = ts[v]
                    x = buf[bases[v], pl.ds((g * UNROLL + u) * 16, 16)]
                    tl = [jnp.maximum(x, t[0])] + list(t[1:])
                    for j in range(TOPK - 1):
                        hi = jnp.maximum(tl[j], tl[j + 1])
                        lo = jnp.minimum(tl[j], tl[j + 1])
                        tl[j], tl[j + 1] = lo, hi
                    tsn.append(tuple(tl))
                ts = tuple(tsn)
            return ts

        ts = lax.fori_loop(0, N // 16 // UNROLL, chunk_step,
                           (((lowest,) * TOPK),) * ROWG)

        thetas = []
        for v in range(ROWG):
            tl = list(ts[v])
            theta = None
            for _ in range(TOPK):
                m = jnp.max(tl[TOPK - 1])
                win1 = jnp.min(jnp.where(tl[TOPK - 1] == m, lane, 16))
                sel = lane == win1
                for j in range(TOPK - 1, 0, -1):
                    tl[j] = jnp.where(sel, tl[j - 1], tl[j])
                tl[0] = jnp.where(sel, lowest, tl[0])
                theta = m
            thetas.append(theta)

        def mask_step(g, carry):
            for u in range(UNROLL):
                for v in range(ROWG):
                    off = (g * UNROLL + u) * 16
                    x = buf[bases[v], pl.ds(off, 16)]
                    buf[bases[v], pl.ds(off, 16)] = jnp.where(
                        x >= thetas[v], x, 0.0)
            return carry

        return lax.fori_loop(0, N // 16 // UNROLL, mask_step, carry)

    row0 = wid * SC_RPW
    pltpu.sync_copy(gs_hbm.at[pl.ds(row0, SC_BATCH), :], buf)
    lax.fori_loop(0, SC_BATCH // ROWG, do_rows, 0)
    pltpu.sync_copy(buf, attn_hbm.at[pl.ds(row0, SC_BATCH), :])


_sc_topk = pl.kernel(
    _sc_topk_body,
    out_type=jax.ShapeDtypeStruct((SC_ROWS, N), jnp.float32),
    mesh=plsc.VectorSubcoreMesh(core_axis_name="c", subcore_axis_name="s"),
    scratch_types=[pltpu.VMEM((SC_BATCH, N), jnp.float32),
                   pltpu.SemaphoreType.DMA],
    compiler_params=pltpu.CompilerParams(use_tc_tiling_on_sc=False,
                                         needs_layout_passes=False),
)


def _horner_body(a1_ref, a2_ref, ah_ref, preds_ref, b2_ref, b1_ref, out_ref):
    dn = (((1,), (0,)), ((), ()))
    a = jnp.concatenate([a1_ref[...], a2_ref[...]], axis=0)
    tmp = preds_ref[...]
    acc = tmp * b2_ref[0, 0]
    for i in range(1, NIT):
        tmp = jax.lax.dot_general(a, tmp, dn, preferred_element_type=jnp.float32)
        tmp = jax.lax.dot_general(a, tmp, dn, preferred_element_type=jnp.float32)
        acc = acc + tmp * b2_ref[0, i]
    ah = ah_ref[...]
    tmp = acc
    acc = tmp * b1_ref[0, 0]
    for i in range(1, NIT):
        tmp = jax.lax.dot_general(ah, tmp, dn, preferred_element_type=jnp.float32)
        acc = acc + tmp * b1_ref[0, i]
    out_ref[...] = acc


def kernel(local_preds, idx, origin_fea, A_hat, Wq_w, Wq_b, Wk_w, Wk_b,
           lin1_w, lin2_w, gumbel):
    f32 = jnp.float32
    bq = Wq_b.reshape(1, F)
    bk = Wk_b.reshape(1, F)

    qs, k = pl.pallas_call(
        _proj_body,
        out_shape=[jax.ShapeDtypeStruct((N, F), f32),
                   jax.ShapeDtypeStruct((N, F), f32)],
    )(origin_fea, Wq_w, bq, Wk_w, bk)

    nblk = N // BLK
    gs = pl.pallas_call(
        _attn_body,
        grid=(nblk,),
        in_specs=[pl.BlockSpec((BLK, F), lambda i: (i, 0)),
                  pl.BlockSpec((N, F), lambda i: (0, 0)),
                  pl.BlockSpec((BLK, N), lambda i: (i, 0))],
        out_specs=pl.BlockSpec((BLK, N), lambda i: (i, 0)),
        out_shape=jax.ShapeDtypeStruct((N, N), f32),
    )(qs, k, gumbel)

    attn_bot = _sc_topk(jax.lax.slice(gs, (SPLIT, 0), (N, N)))

    attn_top = pl.pallas_call(
        _topk_mask_body,
        grid=(SPLIT // BLK,),
        in_specs=[pl.BlockSpec((BLK, N), lambda i: (i, 0))],
        out_specs=pl.BlockSpec((BLK, N), lambda i: (i, 0)),
        out_shape=jax.ShapeDtypeStruct((SPLIT, N), f32),
    )(gs)

    out = pl.pallas_call(
        _horner_body,
        in_specs=[pl.BlockSpec(memory_space=pltpu.MemorySpace.VMEM),
                  pl.BlockSpec(memory_space=pltpu.MemorySpace.VMEM),
                  pl.BlockSpec(memory_space=pltpu.MemorySpace.VMEM),
                  pl.BlockSpec(memory_space=pltpu.MemorySpace.VMEM),
                  pl.BlockSpec(memory_space=pltpu.MemorySpace.SMEM),
                  pl.BlockSpec(memory_space=pltpu.MemorySpace.SMEM)],
        out_shape=jax.ShapeDtypeStruct((N, NNCLS), f32),
    )(attn_top, attn_bot, A_hat, local_preds, lin2_w, lin1_w)
    return out
